# baseline (speedup 1.0000x reference)
import functools

import jax
import jax.numpy as jnp
from jax import lax
from jax.experimental import pallas as pl
from jax.experimental.pallas import tpu as pltpu
from jax.experimental.pallas import tpu_sc as plsc

N = 10000
E = 320000
D = 128
H = 128
C = 64

NC = 2
NS = 16
NW = NC * NS
EPT = E // NW
ROWS_PT = 624
TAIL0 = ROWS_PT * NS
TAIL = N - TAIL0


def _make_seg_sum(w: int, batch: int, with_deg: bool, ring: int = 2):
  nb = EPT // batch
  mesh = plsc.VectorSubcoreMesh(core_axis_name="c", subcore_axis_name="s")
  if with_deg:
    out_type = [jax.ShapeDtypeStruct((NC, N, 128), jnp.float32),
                jax.ShapeDtypeStruct((NC, N, 8), jnp.float32)]
  else:
    out_type = jax.ShapeDtypeStruct((NC, N, w), jnp.float32)

  @functools.partial(
      pl.kernel,
      out_type=out_type,
      mesh=mesh,
      compiler_params=pltpu.CompilerParams(use_tc_tiling_on_sc=False),
      scratch_types=[
          pltpu.VMEM((EPT,), jnp.int32),
          pltpu.VMEM((EPT,), jnp.int32),
          pltpu.VMEM((ring, batch, w), jnp.float32),
          pltpu.SemaphoreType.DMA((ring,)),
          pltpu.VMEM_SHARED((N, w), jnp.float32),
      ],
  )
  def seg(edges_hbm, p_hbm, zeros_hbm, *out_and_scratch):
    if with_deg:
      outf_hbm, outd_hbm, src_v, dst_v, buf, gsem, acc = out_and_scratch
    else:
      outf_hbm, src_v, dst_v, buf, gsem, acc = out_and_scratch
      outd_hbm = None
    c = lax.axis_index("c")
    s = lax.axis_index("s")
    wid = s * NC + c
    r0 = s * ROWS_PT


    pltpu.sync_copy(zeros_hbm.at[pl.ds(r0, ROWS_PT)], acc.at[pl.ds(r0, ROWS_PT)])

    @pl.when(s == NS - 1)
    def _():
      pltpu.sync_copy(zeros_hbm.at[pl.ds(TAIL0, TAIL)],
                      acc.at[pl.ds(TAIL0, TAIL)])

    pltpu.sync_copy(edges_hbm.at[0, pl.ds(wid * EPT, EPT)], src_v)
    pltpu.sync_copy(edges_hbm.at[1, pl.ds(wid * EPT, EPT)], dst_v)
    plsc.subcore_barrier()

    pltpu.async_copy(p_hbm.at[src_v.at[pl.ds(0, batch)]], buf.at[0], gsem.at[0])

    def body(j, carry):
      nx = j + 1

      if ring == 2:
        @pl.when(nx < nb)
        def _():
          pltpu.async_copy(p_hbm.at[src_v.at[pl.ds(nx * batch, batch)]],
                           buf.at[nx % 2], gsem.at[nx % 2])

      pltpu.make_async_copy(p_hbm.at[src_v.at[pl.ds(j * batch, batch)]],
                            buf.at[j % ring], gsem.at[j % ring]).wait()
      pltpu.sync_copy(buf.at[j % ring],
                      acc.at[dst_v.at[pl.ds(j * batch, batch)]], add=True)

      if ring == 1:
        @pl.when(nx < nb)
        def _():
          pltpu.async_copy(p_hbm.at[src_v.at[pl.ds(nx * batch, batch)]],
                           buf.at[0], gsem.at[0])
      return carry

    lax.fori_loop(0, nb, body, 0)

    plsc.subcore_barrier()

    def copy_out(rr, nr):
      if with_deg:
        pltpu.sync_copy(acc.at[pl.ds(rr, nr), pl.ds(0, 128)],
                        outf_hbm.at[c, pl.ds(rr, nr)])
        pltpu.sync_copy(acc.at[pl.ds(rr, nr), pl.ds(128, 8)],
                        outd_hbm.at[c, pl.ds(rr, nr)])
      else:
        pltpu.sync_copy(acc.at[pl.ds(rr, nr)], outf_hbm.at[c, pl.ds(rr, nr)])

    copy_out(r0, ROWS_PT)

    @pl.when(s == NS - 1)
    def _():
      copy_out(TAIL0, TAIL)

  return seg


_seg136 = _make_seg_sum(H + 8, 80, True)
_seg128 = _make_seg_sum(H, 200, False, ring=1)
_seg64 = _make_seg_sum(C, 400, False)

_R = 2000
_G = N // _R


def _mm0_body(x_ref, ws_ref, wn_ref, b_ref, s_ref, p_ref):
  x = x_ref[...]
  s_ref[...] = jnp.dot(x, ws_ref[...],
                       preferred_element_type=jnp.float32) + b_ref[...]
  p_ref[:, :H] = jnp.dot(x, wn_ref[...], preferred_element_type=jnp.float32)
  p_ref[:, H:] = jnp.ones((_R, 8), jnp.float32)


def _stage1_body(s0_ref, g_ref, gd_ref, ws_ref, wn_ref, b_ref,
                 s1_ref, p1_ref, inv_ref):
  deg = jnp.maximum(gd_ref[0, :, 0:1] + gd_ref[1, :, 0:1], 1.0)
  inv = 1.0 / deg
  h = jnp.maximum(s0_ref[...] + (g_ref[0] + g_ref[1]) * inv, 0.0)
  s1_ref[...] = jnp.dot(h, ws_ref[...],
                        preferred_element_type=jnp.float32) + b_ref[...]
  p1_ref[...] = jnp.dot(h, wn_ref[...], preferred_element_type=jnp.float32)
  inv_ref[...] = jnp.broadcast_to(inv, (_R, 8))


def _stage2_body(s1_ref, g_ref, inv_ref, ws_ref, wn_ref, b_ref,
                 s2_ref, p2_ref):
  inv = inv_ref[:, 0:1]
  h = jnp.maximum(s1_ref[...] + (g_ref[0] + g_ref[1]) * inv, 0.0)
  s2_ref[...] = jnp.dot(h, ws_ref[...],
                        preferred_element_type=jnp.float32) + b_ref[...]
  p2_ref[...] = jnp.dot(h, wn_ref[...], preferred_element_type=jnp.float32)


def _stage3_body(s2_ref, g_ref, inv_ref, out_ref):
  inv = inv_ref[:, 0:1]
  out_ref[...] = jnp.maximum(s2_ref[...] + (g_ref[0] + g_ref[1]) * inv, 0.0)


def _rows(w):
  return pl.BlockSpec((_R, w), lambda i: (i, 0))


def _pair(w):
  return pl.BlockSpec((NC, _R, w), lambda i: (0, i, 0))


def _whole(a, b):
  return pl.BlockSpec((a, b), lambda i: (0, 0))


def kernel(x, edge_index, W_self0, W_neigh0, b0, W_self1, W_neigh1, b1,
           W_self2, W_neigh2, b2):
  z136 = jnp.zeros((N, H + 8), jnp.float32)
  z128 = jnp.zeros((N, H), jnp.float32)
  z64 = jnp.zeros((N, C), jnp.float32)

  s0, p0 = pl.pallas_call(
      _mm0_body,
      grid=(_G,),
      in_specs=[_rows(D), _whole(D, H), _whole(D, H), _whole(1, H)],
      out_specs=[_rows(H), _rows(H + 8)],
      out_shape=[jax.ShapeDtypeStruct((N, H), jnp.float32),
                 jax.ShapeDtypeStruct((N, H + 8), jnp.float32)],
  )(x, W_self0, W_neigh0, b0.reshape(1, H))

  g0, g0d = _seg136(edge_index, p0, z136)

  s1, p1, invd = pl.pallas_call(
      _stage1_body,
      grid=(_G,),
      in_specs=[_rows(H), _pair(H), _pair(8), _whole(H, H), _whole(H, H),
                _whole(1, H)],
      out_specs=[_rows(H), _rows(H), _rows(8)],
      out_shape=[jax.ShapeDtypeStruct((N, H), jnp.float32),
                 jax.ShapeDtypeStruct((N, H), jnp.float32),
                 jax.ShapeDtypeStruct((N, 8), jnp.float32)],
  )(s0, g0, g0d, W_self1, W_neigh1, b1.reshape(1, H))

  g1 = _seg128(edge_index, p1, z128)

  s2, p2 = pl.pallas_call(
      _stage2_body,
      grid=(_G,),
      in_specs=[_rows(H), _pair(H), _rows(8), _whole(H, C), _whole(H, C),
                _whole(1, C)],
      out_specs=[_rows(C), _rows(C)],
      out_shape=[jax.ShapeDtypeStruct((N, C), jnp.float32),
                 jax.ShapeDtypeStruct((N, C), jnp.float32)],
  )(s1, g1, invd, W_self2, W_neigh2, b2.reshape(1, C))

  g2 = _seg64(edge_index, p2, z64)

  out = pl.pallas_call(
      _stage3_body,
      grid=(_G,),
      in_specs=[_rows(C), _pair(C), _rows(8)],
      out_specs=_rows(C),
      out_shape=jax.ShapeDtypeStruct((N, C), jnp.float32),
  )(s2, g2, invd)

  return out


# speedup vs baseline: 1.0884x; 1.0884x over previous
import functools

import jax
import jax.numpy as jnp
from jax import lax
from jax.experimental import pallas as pl
from jax.experimental.pallas import tpu as pltpu
from jax.experimental.pallas import tpu_sc as plsc

N = 10000
E = 320000
D = 128
H = 128
C = 64

NC = 2
NS = 16
NW = NC * NS
EPT = E // NW
ROWS_PT = 624
TAIL0 = ROWS_PT * NS
TAIL = N - TAIL0


def _make_seg_sum(w: int, batch: int, with_deg: bool, ring: int = 2):
  nb = EPT // batch
  mesh = plsc.VectorSubcoreMesh(core_axis_name="c", subcore_axis_name="s")
  if with_deg:
    out_type = [jax.ShapeDtypeStruct((NC, N, 128), jnp.float32),
                jax.ShapeDtypeStruct((NC, N, 8), jnp.float32)]
  else:
    out_type = jax.ShapeDtypeStruct((NC, N, w), jnp.float32)

  @functools.partial(
      pl.kernel,
      out_type=out_type,
      mesh=mesh,
      compiler_params=pltpu.CompilerParams(use_tc_tiling_on_sc=False),
      scratch_types=[
          pltpu.VMEM((EPT,), jnp.int32),
          pltpu.VMEM((EPT,), jnp.int32),
          pltpu.VMEM((ring, batch, w), jnp.float32),
          pltpu.SemaphoreType.DMA((ring,)),
          pltpu.VMEM_SHARED((N, w), jnp.float32),
      ],
  )
  def seg(edges_hbm, p_hbm, zeros_hbm, *out_and_scratch):
    if with_deg:
      outf_hbm, outd_hbm, src_v, dst_v, buf, gsem, acc = out_and_scratch
    else:
      outf_hbm, src_v, dst_v, buf, gsem, acc = out_and_scratch
      outd_hbm = None
    c = lax.axis_index("c")
    s = lax.axis_index("s")
    wid = s * NC + c
    r0 = s * ROWS_PT


    pltpu.sync_copy(zeros_hbm.at[pl.ds(r0, ROWS_PT)], acc.at[pl.ds(r0, ROWS_PT)])

    @pl.when(s == NS - 1)
    def _():
      pltpu.sync_copy(zeros_hbm.at[pl.ds(TAIL0, TAIL)],
                      acc.at[pl.ds(TAIL0, TAIL)])

    pltpu.sync_copy(edges_hbm.at[0, pl.ds(wid * EPT, EPT)], src_v)
    pltpu.sync_copy(edges_hbm.at[1, pl.ds(wid * EPT, EPT)], dst_v)
    plsc.subcore_barrier()

    pltpu.async_copy(p_hbm.at[src_v.at[pl.ds(0, batch)]], buf.at[0], gsem.at[0])

    def body(j, carry):
      nx = j + 1

      if ring == 2:
        @pl.when(nx < nb)
        def _():
          pltpu.async_copy(p_hbm.at[src_v.at[pl.ds(nx * batch, batch)]],
                           buf.at[nx % 2], gsem.at[nx % 2])

      pltpu.make_async_copy(p_hbm.at[src_v.at[pl.ds(j * batch, batch)]],
                            buf.at[j % ring], gsem.at[j % ring]).wait()
      pltpu.sync_copy(buf.at[j % ring],
                      acc.at[dst_v.at[pl.ds(j * batch, batch)]], add=True)

      if ring == 1:
        @pl.when(nx < nb)
        def _():
          pltpu.async_copy(p_hbm.at[src_v.at[pl.ds(nx * batch, batch)]],
                           buf.at[0], gsem.at[0])
      return carry

    lax.fori_loop(0, nb, body, 0)

    plsc.subcore_barrier()

    def copy_out(rr, nr):
      if with_deg:
        pltpu.sync_copy(acc.at[pl.ds(rr, nr), pl.ds(0, 128)],
                        outf_hbm.at[c, pl.ds(rr, nr)])
        pltpu.sync_copy(acc.at[pl.ds(rr, nr), pl.ds(128, 8)],
                        outd_hbm.at[c, pl.ds(rr, nr)])
      else:
        pltpu.sync_copy(acc.at[pl.ds(rr, nr)], outf_hbm.at[c, pl.ds(rr, nr)])

    copy_out(r0, ROWS_PT)

    @pl.when(s == NS - 1)
    def _():
      copy_out(TAIL0, TAIL)

  return seg


_seg136 = _make_seg_sum(H + 8, 80, True)
_seg128 = _make_seg_sum(H, 80, False)
_seg64 = _make_seg_sum(C, 400, False)

_R = 2000
_G = N // _R


def _mm0_body(x_ref, ws_ref, wn_ref, b_ref, s_ref, p_ref):
  x = x_ref[...]
  s_ref[...] = jnp.dot(x, ws_ref[...],
                       preferred_element_type=jnp.float32) + b_ref[...]
  p_ref[:, :H] = jnp.dot(x, wn_ref[...], preferred_element_type=jnp.float32)
  p_ref[:, H:] = jnp.ones((_R, 8), jnp.float32)


def _stage1_body(s0_ref, g_ref, gd_ref, ws_ref, wn_ref, b_ref,
                 s1_ref, p1_ref, inv_ref):
  deg = jnp.maximum(gd_ref[0, :, 0:1] + gd_ref[1, :, 0:1], 1.0)
  inv = 1.0 / deg
  h = jnp.maximum(s0_ref[...] + (g_ref[0] + g_ref[1]) * inv, 0.0)
  s1_ref[...] = jnp.dot(h, ws_ref[...],
                        preferred_element_type=jnp.float32) + b_ref[...]
  p1_ref[...] = jnp.dot(h, wn_ref[...], preferred_element_type=jnp.float32)
  inv_ref[...] = jnp.broadcast_to(inv, (_R, 8))


def _stage2_body(s1_ref, g_ref, inv_ref, ws_ref, wn_ref, b_ref,
                 s2_ref, p2_ref):
  inv = inv_ref[:, 0:1]
  h = jnp.maximum(s1_ref[...] + (g_ref[0] + g_ref[1]) * inv, 0.0)
  s2_ref[...] = jnp.dot(h, ws_ref[...],
                        preferred_element_type=jnp.float32) + b_ref[...]
  p2_ref[...] = jnp.dot(h, wn_ref[...], preferred_element_type=jnp.float32)


def _stage3_body(s2_ref, g_ref, inv_ref, out_ref):
  inv = inv_ref[:, 0:1]
  out_ref[...] = jnp.maximum(s2_ref[...] + (g_ref[0] + g_ref[1]) * inv, 0.0)


def _rows(w):
  return pl.BlockSpec((_R, w), lambda i: (i, 0))


def _pair(w):
  return pl.BlockSpec((NC, _R, w), lambda i: (0, i, 0))


def _whole(a, b):
  return pl.BlockSpec((a, b), lambda i: (0, 0))


def kernel(x, edge_index, W_self0, W_neigh0, b0, W_self1, W_neigh1, b1,
           W_self2, W_neigh2, b2):
  z136 = jnp.zeros((N, H + 8), jnp.float32)
  z128 = jnp.zeros((N, H), jnp.float32)
  z64 = jnp.zeros((N, C), jnp.float32)

  s0, p0 = pl.pallas_call(
      _mm0_body,
      grid=(_G,),
      in_specs=[_rows(D), _whole(D, H), _whole(D, H), _whole(1, H)],
      out_specs=[_rows(H), _rows(H + 8)],
      out_shape=[jax.ShapeDtypeStruct((N, H), jnp.float32),
                 jax.ShapeDtypeStruct((N, H + 8), jnp.float32)],
  )(x, W_self0, W_neigh0, b0.reshape(1, H))

  g0, g0d = _seg136(edge_index, p0, z136)

  s1, p1, invd = pl.pallas_call(
      _stage1_body,
      grid=(_G,),
      in_specs=[_rows(H), _pair(H), _pair(8), _whole(H, H), _whole(H, H),
                _whole(1, H)],
      out_specs=[_rows(H), _rows(H), _rows(8)],
      out_shape=[jax.ShapeDtypeStruct((N, H), jnp.float32),
                 jax.ShapeDtypeStruct((N, H), jnp.float32),
                 jax.ShapeDtypeStruct((N, 8), jnp.float32)],
  )(s0, g0, g0d, W_self1, W_neigh1, b1.reshape(1, H))

  g1 = _seg128(edge_index, p1, z128)

  s2, p2 = pl.pallas_call(
      _stage2_body,
      grid=(_G,),
      in_specs=[_rows(H), _pair(H), _rows(8), _whole(H, C), _whole(H, C),
                _whole(1, C)],
      out_specs=[_rows(C), _rows(C)],
      out_shape=[jax.ShapeDtypeStruct((N, C), jnp.float32),
                 jax.ShapeDtypeStruct((N, C), jnp.float32)],
  )(s1, g1, invd, W_self2, W_neigh2, b2.reshape(1, C))

  g2 = _seg64(edge_index, p2, z64)

  out = pl.pallas_call(
      _stage3_body,
      grid=(_G,),
      in_specs=[_rows(C), _pair(C), _rows(8)],
      out_specs=_rows(C),
      out_shape=jax.ShapeDtypeStruct((N, C), jnp.float32),
  )(s2, g2, invd)

  return out


# speedup vs baseline: 1.1464x; 1.0533x over previous
import functools

import jax
import jax.numpy as jnp
from jax import lax
from jax.experimental import pallas as pl
from jax.experimental.pallas import tpu as pltpu
from jax.experimental.pallas import tpu_sc as plsc

N = 10000
E = 320000
D = 128
H = 128
C = 64

NC = 2
NS = 16
NW = NC * NS
EPT = E // NW
ROWS_PT = 624
TAIL0 = ROWS_PT * NS
TAIL = N - TAIL0


def _make_seg_sum(w: int, batch: int, with_deg: bool, ring: int = 2):
  nb = EPT // batch
  mesh = plsc.VectorSubcoreMesh(core_axis_name="c", subcore_axis_name="s")
  if with_deg:
    out_type = [jax.ShapeDtypeStruct((NC, N, 128), jnp.float32),
                jax.ShapeDtypeStruct((NC, N, 8), jnp.float32)]
  else:
    out_type = jax.ShapeDtypeStruct((NC, N, w), jnp.float32)

  @functools.partial(
      pl.kernel,
      out_type=out_type,
      mesh=mesh,
      compiler_params=pltpu.CompilerParams(use_tc_tiling_on_sc=False),
      scratch_types=[
          pltpu.VMEM((EPT,), jnp.int32),
          pltpu.VMEM((EPT,), jnp.int32),
          pltpu.VMEM((ring, batch, w), jnp.float32),
          pltpu.SemaphoreType.DMA((ring,)),
          pltpu.VMEM_SHARED((N, w), jnp.float32),
      ],
  )
  def seg(edges_hbm, p_hbm, zeros_hbm, *out_and_scratch):
    if with_deg:
      outf_hbm, outd_hbm, src_v, dst_v, buf, gsem, acc = out_and_scratch
    else:
      outf_hbm, src_v, dst_v, buf, gsem, acc = out_and_scratch
      outd_hbm = None
    c = lax.axis_index("c")
    s = lax.axis_index("s")
    wid = s * NC + c
    r0 = s * ROWS_PT


    pltpu.sync_copy(zeros_hbm.at[pl.ds(r0, ROWS_PT)], acc.at[pl.ds(r0, ROWS_PT)])

    @pl.when(s == NS - 1)
    def _():
      pltpu.sync_copy(zeros_hbm.at[pl.ds(TAIL0, TAIL)],
                      acc.at[pl.ds(TAIL0, TAIL)])

    pltpu.sync_copy(edges_hbm.at[0, pl.ds(wid * EPT, EPT)], src_v)
    pltpu.sync_copy(edges_hbm.at[1, pl.ds(wid * EPT, EPT)], dst_v)
    plsc.subcore_barrier()

    pltpu.async_copy(p_hbm.at[src_v.at[pl.ds(0, batch)]], buf.at[0], gsem.at[0])

    def body(j, carry):
      nx = j + 1

      if ring == 2:
        @pl.when(nx < nb)
        def _():
          pltpu.async_copy(p_hbm.at[src_v.at[pl.ds(nx * batch, batch)]],
                           buf.at[nx % 2], gsem.at[nx % 2])

      pltpu.make_async_copy(p_hbm.at[src_v.at[pl.ds(j * batch, batch)]],
                            buf.at[j % ring], gsem.at[j % ring]).wait()
      pltpu.sync_copy(buf.at[j % ring],
                      acc.at[dst_v.at[pl.ds(j * batch, batch)]], add=True)

      if ring == 1:
        @pl.when(nx < nb)
        def _():
          pltpu.async_copy(p_hbm.at[src_v.at[pl.ds(nx * batch, batch)]],
                           buf.at[0], gsem.at[0])
      return carry

    lax.fori_loop(0, nb, body, 0)

    plsc.subcore_barrier()

    def copy_out(rr, nr):
      if with_deg:
        pltpu.sync_copy(acc.at[pl.ds(rr, nr), pl.ds(0, 128)],
                        outf_hbm.at[c, pl.ds(rr, nr)])
        pltpu.sync_copy(acc.at[pl.ds(rr, nr), pl.ds(128, 8)],
                        outd_hbm.at[c, pl.ds(rr, nr)])
      else:
        pltpu.sync_copy(acc.at[pl.ds(rr, nr)], outf_hbm.at[c, pl.ds(rr, nr)])

    copy_out(r0, ROWS_PT)

    @pl.when(s == NS - 1)
    def _():
      copy_out(TAIL0, TAIL)

  return seg


NB128 = 78
NCH = 6
CHB = NB128 // NCH
CHW = CHB * 128
TOFF = NB128 * 128 * NW


def _make_seg_sum128(w: int, with_deg: bool):
  mesh = plsc.VectorSubcoreMesh(core_axis_name="c", subcore_axis_name="s")
  if with_deg:
    out_type = [jax.ShapeDtypeStruct((NC, N, 128), jnp.float32),
                jax.ShapeDtypeStruct((NC, N, 8), jnp.float32)]
  else:
    out_type = jax.ShapeDtypeStruct((NC, N, w), jnp.float32)

  @functools.partial(
      pl.kernel,
      out_type=out_type,
      mesh=mesh,
      compiler_params=pltpu.CompilerParams(use_tc_tiling_on_sc=False),
      scratch_types=[
          pltpu.VMEM((2, 2, CHW), jnp.int32),
          pltpu.VMEM((2, 128, w), jnp.float32),
          pltpu.SemaphoreType.DMA((2,)),
          pltpu.SemaphoreType.DMA((2,)),
          pltpu.VMEM_SHARED((N, w), jnp.float32),
      ],
  )
  def seg(edges_hbm, p_hbm, zeros_hbm, *out_and_scratch):
    if with_deg:
      outf_hbm, outd_hbm, idx_v, buf, gsem, isem, acc = out_and_scratch
    else:
      outf_hbm, idx_v, buf, gsem, isem, acc = out_and_scratch
      outd_hbm = None
    c = lax.axis_index("c")
    s = lax.axis_index("s")
    wid = s * NC + c
    r0 = s * ROWS_PT
    off = wid * NB128 * 128

    pltpu.sync_copy(zeros_hbm.at[pl.ds(r0, ROWS_PT)], acc.at[pl.ds(r0, ROWS_PT)])

    @pl.when(s == NS - 1)
    def _():
      pltpu.sync_copy(zeros_hbm.at[pl.ds(TAIL0, TAIL)],
                      acc.at[pl.ds(TAIL0, TAIL)])

    def load_chunk_start(g, slot):
      pltpu.async_copy(edges_hbm.at[0, pl.ds(off + g * CHW, CHW)],
                       idx_v.at[slot, 0], isem.at[slot])
      pltpu.async_copy(edges_hbm.at[1, pl.ds(off + g * CHW, CHW)],
                       idx_v.at[slot, 1], isem.at[slot])

    def load_chunk_wait(g, slot):
      pltpu.make_async_copy(edges_hbm.at[0, pl.ds(off + g * CHW, CHW)],
                            idx_v.at[slot, 0], isem.at[slot]).wait()
      pltpu.make_async_copy(edges_hbm.at[1, pl.ds(off + g * CHW, CHW)],
                            idx_v.at[slot, 1], isem.at[slot]).wait()

    def gidx(slot, j):
      return idx_v.at[slot, 0, pl.ds(j * 128, 128)]

    def sidx(slot, j):
      return idx_v.at[slot, 1, pl.ds(j * 128, 128)]

    def gather_start(slot, j, b):
      pltpu.async_copy(p_hbm.at[gidx(slot, j)], buf.at[b % 2], gsem.at[b % 2])

    def gather_wait(slot, j, b):
      pltpu.make_async_copy(p_hbm.at[gidx(slot, j)], buf.at[b % 2],
                            gsem.at[b % 2]).wait()

    load_chunk_start(0, 0)
    load_chunk_wait(0, 0)
    gather_start(0, 0, 0)
    load_chunk_start(1, 1)
    plsc.subcore_barrier()

    for g in range(NCH):
      cs = g % 2

      def body(j, carry, g=g, cs=cs):
        b = g * CHB + j

        @pl.when(j < CHB - 1)
        def _():
          gather_start(cs, j + 1, b + 1)

        if g < NCH - 1:
          @pl.when(j == CHB - 1)
          def _():
            load_chunk_wait(g + 1, 1 - cs)
            gather_start(1 - cs, 0, b + 1)

        gather_wait(cs, j, b)
        pltpu.sync_copy(buf.at[b % 2], acc.at[sidx(cs, j)], add=True)
        return carry

      lax.fori_loop(0, CHB, body, 0)
      if g + 2 < NCH:
        load_chunk_start(g + 2, cs)

    @pl.when(wid < 4)
    def _():
      toff = TOFF + wid * 128
      pltpu.sync_copy(edges_hbm.at[0, pl.ds(toff, 128)],
                      idx_v.at[0, 0, pl.ds(0, 128)])
      pltpu.sync_copy(edges_hbm.at[1, pl.ds(toff, 128)],
                      idx_v.at[0, 1, pl.ds(0, 128)])
      gather_start(0, 0, 0)
      gather_wait(0, 0, 0)
      pltpu.sync_copy(buf.at[0], acc.at[sidx(0, 0)], add=True)

    plsc.subcore_barrier()

    def copy_out(rr, nr):
      if with_deg:
        pltpu.sync_copy(acc.at[pl.ds(rr, nr), pl.ds(0, 128)],
                        outf_hbm.at[c, pl.ds(rr, nr)])
        pltpu.sync_copy(acc.at[pl.ds(rr, nr), pl.ds(128, 8)],
                        outd_hbm.at[c, pl.ds(rr, nr)])
      else:
        pltpu.sync_copy(acc.at[pl.ds(rr, nr)], outf_hbm.at[c, pl.ds(rr, nr)])

    copy_out(r0, ROWS_PT)

    @pl.when(s == NS - 1)
    def _():
      copy_out(TAIL0, TAIL)

  return seg


_seg136 = _make_seg_sum128(H + 8, True)
_seg128 = _make_seg_sum128(H, False)
_seg64 = _make_seg_sum(C, 400, False)

_R = 2000
_G = N // _R


def _mm0_body(x_ref, ws_ref, wn_ref, b_ref, s_ref, p_ref):
  x = x_ref[...]
  s_ref[...] = jnp.dot(x, ws_ref[...],
                       preferred_element_type=jnp.float32) + b_ref[...]
  p_ref[:, :H] = jnp.dot(x, wn_ref[...], preferred_element_type=jnp.float32)
  p_ref[:, H:] = jnp.ones((_R, 8), jnp.float32)


def _stage1_body(s0_ref, g_ref, gd_ref, ws_ref, wn_ref, b_ref,
                 s1_ref, p1_ref, inv_ref):
  deg = jnp.maximum(gd_ref[0, :, 0:1] + gd_ref[1, :, 0:1], 1.0)
  inv = 1.0 / deg
  h = jnp.maximum(s0_ref[...] + (g_ref[0] + g_ref[1]) * inv, 0.0)
  s1_ref[...] = jnp.dot(h, ws_ref[...],
                        preferred_element_type=jnp.float32) + b_ref[...]
  p1_ref[...] = jnp.dot(h, wn_ref[...], preferred_element_type=jnp.float32)
  inv_ref[...] = jnp.broadcast_to(inv, (_R, 8))


def _stage2_body(s1_ref, g_ref, inv_ref, ws_ref, wn_ref, b_ref,
                 s2_ref, p2_ref):
  inv = inv_ref[:, 0:1]
  h = jnp.maximum(s1_ref[...] + (g_ref[0] + g_ref[1]) * inv, 0.0)
  s2_ref[...] = jnp.dot(h, ws_ref[...],
                        preferred_element_type=jnp.float32) + b_ref[...]
  p2_ref[...] = jnp.dot(h, wn_ref[...], preferred_element_type=jnp.float32)


def _stage3_body(s2_ref, g_ref, inv_ref, out_ref):
  inv = inv_ref[:, 0:1]
  out_ref[...] = jnp.maximum(s2_ref[...] + (g_ref[0] + g_ref[1]) * inv, 0.0)


def _rows(w):
  return pl.BlockSpec((_R, w), lambda i: (i, 0))


def _pair(w):
  return pl.BlockSpec((NC, _R, w), lambda i: (0, i, 0))


def _whole(a, b):
  return pl.BlockSpec((a, b), lambda i: (0, 0))


def kernel(x, edge_index, W_self0, W_neigh0, b0, W_self1, W_neigh1, b1,
           W_self2, W_neigh2, b2):
  z136 = jnp.zeros((N, H + 8), jnp.float32)
  z128 = jnp.zeros((N, H), jnp.float32)
  z64 = jnp.zeros((N, C), jnp.float32)

  s0, p0 = pl.pallas_call(
      _mm0_body,
      grid=(_G,),
      in_specs=[_rows(D), _whole(D, H), _whole(D, H), _whole(1, H)],
      out_specs=[_rows(H), _rows(H + 8)],
      out_shape=[jax.ShapeDtypeStruct((N, H), jnp.float32),
                 jax.ShapeDtypeStruct((N, H + 8), jnp.float32)],
  )(x, W_self0, W_neigh0, b0.reshape(1, H))

  g0, g0d = _seg136(edge_index, p0, z136)

  s1, p1, invd = pl.pallas_call(
      _stage1_body,
      grid=(_G,),
      in_specs=[_rows(H), _pair(H), _pair(8), _whole(H, H), _whole(H, H),
                _whole(1, H)],
      out_specs=[_rows(H), _rows(H), _rows(8)],
      out_shape=[jax.ShapeDtypeStruct((N, H), jnp.float32),
                 jax.ShapeDtypeStruct((N, H), jnp.float32),
                 jax.ShapeDtypeStruct((N, 8), jnp.float32)],
  )(s0, g0, g0d, W_self1, W_neigh1, b1.reshape(1, H))

  g1 = _seg128(edge_index, p1, z128)

  s2, p2 = pl.pallas_call(
      _stage2_body,
      grid=(_G,),
      in_specs=[_rows(H), _pair(H), _rows(8), _whole(H, C), _whole(H, C),
                _whole(1, C)],
      out_specs=[_rows(C), _rows(C)],
      out_shape=[jax.ShapeDtypeStruct((N, C), jnp.float32),
                 jax.ShapeDtypeStruct((N, C), jnp.float32)],
  )(s1, g1, invd, W_self2, W_neigh2, b2.reshape(1, C))

  g2 = _seg64(edge_index, p2, z64)

  out = pl.pallas_call(
      _stage3_body,
      grid=(_G,),
      in_specs=[_rows(C), _pair(C), _rows(8)],
      out_specs=_rows(C),
      out_shape=jax.ShapeDtypeStruct((N, C), jnp.float32),
  )(s2, g2, invd)

  return out
